# Initial kernel scaffold; baseline (speedup 1.0000x reference)
import jax
import jax.numpy as jnp
from jax.experimental import pallas as pl


def kernel(x, edge_index, w_in1, b_in1, w_in2, b_in2, w_in3, b_in3, wq, bq, wk, bk, wv, bv, ws, bs, w_o1, b_o1, w_o2, b_o2, w_o3, b_o3):
    raise NotImplementedError("write your pallas kernel here")


# speedup vs baseline: 19.0927x; 19.0927x over previous
import functools

import jax
import jax.numpy as jnp
import numpy as np
from jax import lax
from jax.experimental import pallas as pl
from jax.experimental.pallas import tpu as pltpu
from jax.experimental.pallas import tpu_sc as plsc

_CH = 80
_NW = 32
_HEADS = 8
_C = 32
_ED = 256


def _head_onehot(ncols, nheads, transpose=False):
    if transpose:
        r = lax.broadcasted_iota(jnp.int32, (nheads, ncols), 1)
        c = lax.broadcasted_iota(jnp.int32, (nheads, ncols), 0)
    else:
        r = lax.broadcasted_iota(jnp.int32, (ncols, nheads), 0)
        c = lax.broadcasted_iota(jnp.int32, (ncols, nheads), 1)
    return (r // _C == c).astype(jnp.float32)


def _dense(a, w, b, relu, bm):
    m, k = a.shape
    n = w.shape[1]

    def kern(a_ref, w_ref, b_ref, o_ref):
        r = jnp.dot(a_ref[...], w_ref[...], preferred_element_type=jnp.float32)
        r = r + b_ref[...]
        o_ref[...] = jnp.maximum(r, 0.0) if relu else r

    return pl.pallas_call(
        kern,
        grid=(m // bm,),
        in_specs=[pl.BlockSpec((bm, k), lambda i: (i, 0)),
                  pl.BlockSpec((k, n), lambda i: (0, 0)),
                  pl.BlockSpec((1, n), lambda i: (0, 0))],
        out_specs=pl.BlockSpec((bm, n), lambda i: (i, 0)),
        out_shape=jax.ShapeDtypeStruct((m, n), jnp.float32),
    )(a, w, b.reshape(1, n))


def _proj(h, w4, b4, bm):
    m = h.shape[0]

    def kern(h_ref, w_ref, b_ref, q_ref, k_ref, v_ref, s_ref):
        p = jnp.dot(h_ref[...], w_ref[...], preferred_element_type=jnp.float32)
        p = p + b_ref[...]
        q_ref[...] = p[:, 0:256]
        k_ref[...] = p[:, 256:512]
        v_ref[...] = p[:, 512:768]
        s_ref[...] = p[:, 768:1024]

    shp = jax.ShapeDtypeStruct((m, _ED), jnp.float32)
    return pl.pallas_call(
        kern,
        grid=(m // bm,),
        in_specs=[pl.BlockSpec((bm, _ED), lambda i: (i, 0)),
                  pl.BlockSpec((_ED, 4 * _ED), lambda i: (0, 0)),
                  pl.BlockSpec((1, 4 * _ED), lambda i: (0, 0))],
        out_specs=[pl.BlockSpec((bm, _ED), lambda i: (i, 0))] * 4,
        out_shape=[shp] * 4,
    )(h, w4, b4.reshape(1, 4 * _ED))


def _kmax(k):
    m = k.shape[0]

    def kern(k_ref, o_ref):
        kk = k_ref[...]
        g = _head_onehot(_ED, _HEADS)
        kn2 = jnp.dot(kk * kk, g, preferred_element_type=jnp.float32)
        o_ref[...] = jnp.sqrt(jnp.max(kn2, axis=0, keepdims=True) / float(_C))

    return pl.pallas_call(
        kern,
        out_shape=jax.ShapeDtypeStruct((1, _HEADS), jnp.float32),
    )(k)


def _edge_math(qd, ks, vs, kmaxs, be):
    etot = qd.shape[0]
    inv = 1.0 / float(np.sqrt(_C))

    def kern(qd_ref, ks_ref, vs_ref, km_ref, w0_ref, w1_ref, ex_ref):
        g = _head_onehot(_ED, _HEADS)
        gt = _head_onehot(_ED, _HEADS, transpose=True)
        q = qd_ref[...]
        k = ks_ref[...]
        score = jnp.dot(q * k, g, preferred_element_type=jnp.float32) * inv
        qn2 = jnp.dot(q * q, g, preferred_element_type=jnp.float32)
        mbound = jnp.sqrt(qn2) * km_ref[...]
        ex = jnp.exp(score - mbound)
        exf = jnp.dot(ex, gt, preferred_element_type=jnp.float32)
        wfull = vs_ref[...] * exf
        w0_ref[...] = wfull[:, 0:128]
        w1_ref[...] = wfull[:, 128:256]
        r16 = lax.broadcasted_iota(jnp.int32, (_HEADS, 128), 0)
        c16 = lax.broadcasted_iota(jnp.int32, (_HEADS, 128), 1)
        g16 = (c16 // 16 == r16).astype(jnp.float32)
        ex_ref[...] = jnp.dot(ex, g16, preferred_element_type=jnp.float32)

    return pl.pallas_call(
        kern,
        grid=(etot // be,),
        in_specs=[pl.BlockSpec((be, _ED), lambda i: (i, 0)),
                  pl.BlockSpec((be, _ED), lambda i: (i, 0)),
                  pl.BlockSpec((be, _ED), lambda i: (i, 0)),
                  pl.BlockSpec((1, _HEADS), lambda i: (0, 0))],
        out_specs=[pl.BlockSpec((be, 128), lambda i: (i, 0)),
                   pl.BlockSpec((be, 128), lambda i: (i, 0)),
                   pl.BlockSpec((be, 128), lambda i: (i, 0))],
        out_shape=[jax.ShapeDtypeStruct((etot, 128), jnp.float32),
                   jax.ShapeDtypeStruct((etot, 128), jnp.float32),
                   jax.ShapeDtypeStruct((etot, 128), jnp.float32)],
    )(qd, ks, vs, kmaxs)


def _combine(out0, out1, den0, den1, s, bm):
    m = s.shape[0]

    def kern(o0_ref, o1_ref, d0_ref, d1_ref, s_ref, h_ref):
        r = lax.broadcasted_iota(jnp.int32, (128, _ED), 0)
        c = lax.broadcasted_iota(jnp.int32, (128, _ED), 1)
        realign = jnp.where(r // 16 == c // 32, 1.0 / 16.0, 0.0)
        d = d0_ref[...] + d1_ref[...]
        d = jnp.dot(d, realign, preferred_element_type=jnp.float32) + 1e-16
        attn = jnp.concatenate([o0_ref[...], o1_ref[...]], axis=1)
        h_ref[...] = attn / d + s_ref[...]

    return pl.pallas_call(
        kern,
        grid=(m // bm,),
        in_specs=[pl.BlockSpec((bm, 128), lambda i: (i, 0)),
                  pl.BlockSpec((bm, 128), lambda i: (i, 0)),
                  pl.BlockSpec((bm, 128), lambda i: (i, 0)),
                  pl.BlockSpec((bm, 128), lambda i: (i, 0)),
                  pl.BlockSpec((bm, _ED), lambda i: (i, 0))],
        out_specs=pl.BlockSpec((bm, _ED), lambda i: (i, 0)),
        out_shape=jax.ShapeDtypeStruct((m, _ED), jnp.float32),
    )(out0, out1, den0, den1, s)


def _final(h, w, b, bm):
    m = h.shape[0]
    n = w.shape[1]

    def kern(h_ref, w_ref, b_ref, o_ref):
        o = jnp.dot(h_ref[...], w_ref[...], preferred_element_type=jnp.float32)
        o = o + b_ref[...]
        norm = jnp.sqrt(jnp.sum(o * o, axis=1, keepdims=True))
        o_ref[...] = o / jnp.maximum(norm, 1e-12)

    return pl.pallas_call(
        kern,
        grid=(m // bm,),
        in_specs=[pl.BlockSpec((bm, _ED), lambda i: (i, 0)),
                  pl.BlockSpec((_ED, n), lambda i: (0, 0)),
                  pl.BlockSpec((1, n), lambda i: (0, 0))],
        out_specs=pl.BlockSpec((bm, n), lambda i: (i, 0)),
        out_shape=jax.ShapeDtypeStruct((m, n), jnp.float32),
    )(h, w, b.reshape(1, n))


def _gather3(q, k, v, dst, src):
    etot = dst.shape[0]
    per_w = etot // _NW
    nch = per_w // _CH
    mesh = plsc.VectorSubcoreMesh(core_axis_name="c", subcore_axis_name="s")
    oshp = jax.ShapeDtypeStruct((etot, _ED), jnp.float32)

    @functools.partial(
        pl.kernel, mesh=mesh,
        out_type=[oshp, oshp, oshp],
        scratch_types=[pltpu.VMEM((_CH,), jnp.int32),
                       pltpu.VMEM((_CH,), jnp.int32),
                       pltpu.VMEM((_CH, _ED), jnp.float32),
                       pltpu.VMEM((_CH, _ED), jnp.float32),
                       pltpu.VMEM((_CH, _ED), jnp.float32),
                       pltpu.SemaphoreType.DMA,
                       pltpu.SemaphoreType.DMA,
                       pltpu.SemaphoreType.DMA])
    def kern(q_hbm, k_hbm, v_hbm, dst_hbm, src_hbm, qd_hbm, ks_hbm, vs_hbm,
             di_v, si_v, qbuf, kbuf, vbuf, sem1, sem2, sem3):
        wid = lax.axis_index("s") * 2 + lax.axis_index("c")
        base0 = wid * per_w

        def body(j, carry):
            base = base0 + j * _CH
            pltpu.sync_copy(dst_hbm.at[pl.ds(base, _CH)], di_v)
            pltpu.sync_copy(src_hbm.at[pl.ds(base, _CH)], si_v)
            c1 = pltpu.async_copy(q_hbm.at[di_v], qbuf, sem1)
            c2 = pltpu.async_copy(k_hbm.at[si_v], kbuf, sem2)
            c3 = pltpu.async_copy(v_hbm.at[si_v], vbuf, sem3)
            c1.wait()
            c2.wait()
            c3.wait()
            pltpu.sync_copy(qbuf, qd_hbm.at[pl.ds(base, _CH)])
            pltpu.sync_copy(kbuf, ks_hbm.at[pl.ds(base, _CH)])
            pltpu.sync_copy(vbuf, vs_hbm.at[pl.ds(base, _CH)])
            return carry

        lax.fori_loop(0, nch, body, 0)

    return kern(q, k, v, dst, src)


def _scatter(w0, w1, ex, dst, np_):
    etot = dst.shape[0]
    per_s = etot // 16
    nch = per_s // _CH
    per_s2 = etot // 32
    nch2 = per_s2 // _CH
    rows_per_sub = np_ // 16
    mesh = plsc.VectorSubcoreMesh(core_axis_name="c", subcore_axis_name="s")
    zrows = jnp.zeros((16, 128), jnp.float32)
    oshp = jax.ShapeDtypeStruct((np_, 128), jnp.float32)

    @functools.partial(
        pl.kernel, mesh=mesh,
        out_type=[oshp, oshp, oshp, oshp],
        scratch_types=[pltpu.VMEM((_CH,), jnp.int32),
                       pltpu.VMEM((_CH, 128), jnp.float32),
                       pltpu.VMEM((16, 128), jnp.float32),
                       pltpu.VMEM_SHARED((np_, 128), jnp.float32)])
    def kern(w0_hbm, w1_hbm, ex_hbm, dst_hbm, z_hbm,
             out0_hbm, out1_hbm, den0_hbm, den1_hbm,
             idx_v, wbuf, zbuf, acc_sh):
        cc = lax.axis_index("c")
        ss = lax.axis_index("s")
        row0 = ss * rows_per_sub

        pltpu.sync_copy(z_hbm, zbuf)

        def zero_acc():
            def zbody(t, carry):
                pltpu.sync_copy(zbuf, acc_sh.at[pl.ds(row0 + t * 16, 16)])
                return carry
            lax.fori_loop(0, rows_per_sub // 16, zbody, 0)

        def accum(src_hbm, base0, nsteps):
            def body(j, carry):
                base = base0 + j * _CH
                pltpu.sync_copy(dst_hbm.at[pl.ds(base, _CH)], idx_v)
                pltpu.sync_copy(src_hbm.at[pl.ds(base, _CH)], wbuf)
                pltpu.sync_copy(wbuf, acc_sh.at[idx_v], add=True)
                return carry
            lax.fori_loop(0, nsteps, body, 0)

        def dump(dst_hbm_out):
            pltpu.sync_copy(acc_sh.at[pl.ds(row0, rows_per_sub)],
                            dst_hbm_out.at[pl.ds(row0, rows_per_sub)])

        zero_acc()
        plsc.subcore_barrier()

        @pl.when(cc == 0)
        def _():
            accum(w0_hbm, ss * per_s, nch)

        @pl.when(cc == 1)
        def _():
            accum(w1_hbm, ss * per_s, nch)

        plsc.subcore_barrier()

        @pl.when(cc == 0)
        def _():
            dump(out0_hbm)

        @pl.when(cc == 1)
        def _():
            dump(out1_hbm)

        plsc.subcore_barrier()

        zero_acc()
        plsc.subcore_barrier()

        @pl.when(cc == 0)
        def _():
            accum(ex_hbm, ss * per_s2, nch2)

        @pl.when(cc == 1)
        def _():
            accum(ex_hbm, etot // 2 + ss * per_s2, nch2)

        plsc.subcore_barrier()

        @pl.when(cc == 0)
        def _():
            dump(den0_hbm)

        @pl.when(cc == 1)
        def _():
            dump(den1_hbm)

    return kern(w0, w1, ex, dst, zrows)


def kernel(x, edge_index, w_in1, b_in1, w_in2, b_in2, w_in3, b_in3,
           wq, bq, wk, bk, wv, bv, ws, bs,
           w_o1, b_o1, w_o2, b_o2, w_o3, b_o3):
    n = x.shape[0]
    npad = ((n + 511) // 512) * 512
    nlayers = wq.shape[0]
    bm = 512

    xp = jnp.pad(x, ((0, npad - n), (0, 0)))
    src = edge_index[0]
    dst = edge_index[1]

    h = _dense(xp, w_in1, b_in1, True, bm)
    h = _dense(h, w_in2, b_in2, True, bm)
    h = _dense(h, w_in3, b_in3, True, bm)

    for l in range(nlayers):
        w4 = jnp.concatenate([wq[l], wk[l], wv[l], ws[l]], axis=1)
        b4 = jnp.concatenate([bq[l], bk[l], bv[l], bs[l]])
        q, k, v, s = _proj(h, w4, b4, bm)
        kmaxs = _kmax(k)
        qd, ksg, vsg = _gather3(q, k, v, dst, src)
        w0, w1, ex = _edge_math(qd, ksg, vsg, kmaxs, 1000)
        out0, out1, den0, den1 = _scatter(w0, w1, ex, dst, npad)
        h = _combine(out0, out1, den0, den1, s, bm)

    h = _dense(h, w_o1, b_o1, True, bm)
    h = _dense(h, w_o2, b_o2, True, bm)
    o = _final(h, w_o3, b_o3, bm)
    return o[:n]
